# baseline (speedup 1.0000x reference)
import functools

import jax
import jax.numpy as jnp
from jax import lax
from jax.experimental import pallas as pl
from jax.experimental.pallas import tpu as pltpu
from jax.experimental.pallas import tpu_sc as plsc

BATCH = 1024
SEQLEN = 200
EMBED = 64
LANES = 16
NC = 2
NS = 16
NW = NC * NS
BPW = BATCH // NW
SC_CH = 8
NCHUNK = SEQLEN // SC_CH
RPC = SC_CH * BPW


def _sc_embed(inp_n, table, pos):
    mesh = plsc.VectorSubcoreMesh(core_axis_name="c", subcore_axis_name="s")

    @functools.partial(
        pl.kernel,
        out_type=jax.ShapeDtypeStruct((SEQLEN, 8, 8, 8, 128), jnp.float32),
        mesh=mesh,
        scratch_types=[
            pltpu.VMEM((NCHUNK, 1, SC_CH, BPW), jnp.int32),
            pltpu.VMEM((NCHUNK, 2, 128), jnp.int32),
            pltpu.VMEM((RPC, EMBED), jnp.float32),
            pltpu.VMEM((RPC, EMBED), jnp.float32),
            pltpu.VMEM((SC_CH, 8, 1, 8, BPW + 1), jnp.float32),
            pltpu.VMEM((SC_CH, 8, 1, 8, BPW + 1), jnp.float32),
            pltpu.VMEM((SEQLEN, EMBED), jnp.float32),
            pltpu.SemaphoreType.DMA,
            pltpu.SemaphoreType.DMA,
        ],
        compiler_params=pltpu.CompilerParams(
            use_tc_tiling_on_sc=False, needs_layout_passes=False),
    )
    def k(inp_hbm, table_hbm, pos_hbm, out_hbm,
          idx_v, idx_f, grows0, grows1, st0, st1, pos_v, gsem, ssem):
        wid = lax.axis_index("s") * NC + lax.axis_index("c")
        bblk = wid // 4
        bsub0 = (wid % 4) * BPW
        grows = (grows0, grows1)
        st = (st0, st1)

        iota = jax.lax.iota(jnp.int32, LANES)
        dsub_i = iota & 7
        zero_i = jnp.zeros((LANES,), jnp.int32)
        dblk_i = [(iota + LANES * j) >> 3 for j in range(EMBED // LANES)]

        def out_slice(c):
            return out_hbm.at[pl.ds(SC_CH * c, SC_CH), :, pl.ds(bblk, 1), :,
                              pl.ds(bsub0, BPW)]

        pltpu.sync_copy(pos_hbm, pos_v)
        pltpu.sync_copy(
            inp_hbm.at[:, pl.ds(bblk, 1), :, pl.ds(bsub0, BPW)], idx_v)

        def repack(c, carry):
            for sl in range(SC_CH):
                for h2 in range(BPW // LANES):
                    flat = sl * BPW + h2 * LANES
                    idx_f[c, flat // 128, pl.ds(flat % 128, LANES)] = (
                        idx_v[c, 0, sl, pl.ds(h2 * LANES, LANES)])
            return carry

        lax.fori_loop(0, NCHUNK, repack, 0)

        def compute(c, p):
            g, s_buf = grows[p], st[p]

            def sbody(sl, carry):
                pos4 = [pos_v[SC_CH * c + sl, pl.ds(LANES * j, LANES)]
                        for j in range(EMBED // LANES)]
                sl_i = jnp.full((LANES,), sl, jnp.int32)

                @plsc.parallel_loop(0, BPW, 1, unroll=8)
                def bbody(b):
                    b_i = jnp.full((LANES,), b, jnp.int32)
                    for j in range(EMBED // LANES):
                        v = g[sl * BPW + b, pl.ds(LANES * j, LANES)] + pos4[j]
                        plsc.store_scatter(
                            s_buf, [sl_i, dblk_i[j], zero_i, dsub_i, b_i], v)

                return carry

            lax.fori_loop(0, SC_CH, sbody, 0)

        def fire_gathers(c, p):
            return [pltpu.async_copy(table_hbm.at[idx_f.at[c, h]],
                                     grows[p].at[pl.ds(h * 128, 128)], gsem)
                    for h in range(2)]

        gd = {0: fire_gathers(0, 0)}
        sd = {}
        for c in range(NCHUNK):
            p = c % 2
            if c + 1 < NCHUNK:
                gd[c + 1] = fire_gathers(c + 1, 1 - p)
            for d in gd[c]:
                d.wait()
            if c >= 2:
                sd[c - 2].wait()
            compute(c, p)
            sd[c] = pltpu.async_copy(
                st[p].at[:, :, :, :, pl.ds(0, BPW)], out_slice(c), ssem)
        sd[NCHUNK - 2].wait()
        sd[NCHUNK - 1].wait()

    return k(inp_n, table, pos)


def kernel(inp, embedding_matrix, position_embedding):
    inp_n = inp.astype(jnp.int32).T.reshape(25, 8, 8, 128).transpose(0, 2, 1, 3)
    inp_n = lax.optimization_barrier(inp_n)
    out5 = _sc_embed(inp_n, embedding_matrix, position_embedding)
    t = out5.transpose(2, 4, 0, 1, 3)
    t = lax.optimization_barrier(t)
    return t.reshape(BATCH, SEQLEN, EMBED)


# speedup vs baseline: 1.1855x; 1.1855x over previous
import functools

import jax
import jax.numpy as jnp
from jax import lax
from jax.experimental import pallas as pl
from jax.experimental.pallas import tpu as pltpu
from jax.experimental.pallas import tpu_sc as plsc

BATCH = 1024
SEQLEN = 200
EMBED = 64
LANES = 16
NC = 2
NS = 16
NW = NC * NS
BPW = BATCH // NW
SC_CH = 8
NCHUNK = SEQLEN // SC_CH
RPC = SC_CH * BPW


def _sc_embed(inp_n, table, pos):
    mesh = plsc.VectorSubcoreMesh(core_axis_name="c", subcore_axis_name="s")

    @functools.partial(
        pl.kernel,
        out_type=jax.ShapeDtypeStruct((SEQLEN, 8, 8, 8, 128), jnp.float32),
        mesh=mesh,
        scratch_types=[
            pltpu.VMEM((NCHUNK, 1, SC_CH, BPW), jnp.int32),
            pltpu.VMEM((NCHUNK, 2, 128), jnp.int32),
            pltpu.VMEM((RPC, EMBED), jnp.float32),
            pltpu.VMEM((RPC, EMBED), jnp.float32),
            pltpu.VMEM((RPC, EMBED), jnp.float32),
            pltpu.VMEM((SC_CH, 8, 1, 8, BPW + 1), jnp.float32),
            pltpu.VMEM((SC_CH, 8, 1, 8, BPW + 1), jnp.float32),
            pltpu.VMEM((SEQLEN, EMBED), jnp.float32),
            pltpu.SemaphoreType.DMA,
            pltpu.SemaphoreType.DMA,
        ],
        compiler_params=pltpu.CompilerParams(
            use_tc_tiling_on_sc=False, needs_layout_passes=False),
    )
    def k(inp_hbm, table_hbm, pos_hbm, out_hbm,
          idx_v, idx_f, grows0, grows1, grows2, st0, st1, pos_v, gsem, ssem):
        wid = lax.axis_index("s") * NC + lax.axis_index("c")
        bblk = wid // 4
        bsub0 = (wid % 4) * BPW
        grows = (grows0, grows1, grows2)
        st = (st0, st1)

        iota = jax.lax.iota(jnp.int32, LANES)
        dsub_i = iota & 7
        zero_i = jnp.zeros((LANES,), jnp.int32)
        dblk_i = [(iota + LANES * j) >> 3 for j in range(EMBED // LANES)]

        def out_slice(c):
            return out_hbm.at[pl.ds(SC_CH * c, SC_CH), :, pl.ds(bblk, 1), :,
                              pl.ds(bsub0, BPW)]

        pltpu.sync_copy(pos_hbm, pos_v)
        pltpu.sync_copy(
            inp_hbm.at[:, pl.ds(bblk, 1), :, pl.ds(bsub0, BPW)], idx_v)

        def repack(c, carry):
            for sl in range(SC_CH):
                for h2 in range(BPW // LANES):
                    flat = sl * BPW + h2 * LANES
                    idx_f[c, flat // 128, pl.ds(flat % 128, LANES)] = (
                        idx_v[c, 0, sl, pl.ds(h2 * LANES, LANES)])
            return carry

        lax.fori_loop(0, NCHUNK, repack, 0)

        def compute(c, gp, sp):
            g, s_buf = grows[gp], st[sp]

            def sbody(sl, carry):
                pos4 = [pos_v[SC_CH * c + sl, pl.ds(LANES * j, LANES)]
                        for j in range(EMBED // LANES)]
                sl_i = jnp.full((LANES,), sl, jnp.int32)

                @plsc.parallel_loop(0, BPW, 1, unroll=4)
                def bbody(b):
                    b_i = jnp.full((LANES,), b, jnp.int32)
                    for j in range(EMBED // LANES):
                        v = g[sl * BPW + b, pl.ds(LANES * j, LANES)] + pos4[j]
                        plsc.store_scatter(
                            s_buf, [sl_i, dblk_i[j], zero_i, dsub_i, b_i], v)

                return carry

            lax.fori_loop(0, SC_CH, sbody, 0)

        def fire_gathers(c, p):
            return [pltpu.async_copy(table_hbm.at[idx_f.at[c, h]],
                                     grows[p].at[pl.ds(h * 128, 128)], gsem)
                    for h in range(2)]

        gd = {0: fire_gathers(0, 0), 1: fire_gathers(1, 1)}
        sd = {}
        for c in range(NCHUNK):
            if c + 2 < NCHUNK:
                gd[c + 2] = fire_gathers(c + 2, (c + 2) % 3)
            for d in gd[c]:
                d.wait()
            if c >= 2:
                sd[c - 2].wait()
            compute(c, c % 3, c % 2)
            sd[c] = pltpu.async_copy(
                st[c % 2].at[:, :, :, :, pl.ds(0, BPW)], out_slice(c), ssem)
        sd[NCHUNK - 2].wait()
        sd[NCHUNK - 1].wait()

    return k(inp_n, table, pos)


def kernel(inp, embedding_matrix, position_embedding):
    inp_n = inp.astype(jnp.int32).T.reshape(25, 8, 8, 128).transpose(0, 2, 1, 3)
    inp_n = lax.optimization_barrier(inp_n)
    out5 = _sc_embed(inp_n, embedding_matrix, position_embedding)
    t = out5.transpose(2, 4, 0, 1, 3)
    t = lax.optimization_barrier(t)
    return t.reshape(BATCH, SEQLEN, EMBED)
